# baseline (device time: 507287 ns/iter reference)
import numpy as np

import jax
import jax.numpy as jnp
from jax import lax
from jax.experimental import pallas as pl
from jax.experimental.pallas import tpu as pltpu

N_DEV = 8
EXPECTED_PI = np.array([3, 4, 5, 6, 7, 0, 1, 2], dtype=np.int32)

GR = 256
G_PER_CHUNK = 4
N_OWN = 16
N_FWD = 20

FLOW_PATHS = {
    0: [([0, 1, 2, 3], 1), ([0, 3], 3)],
    1: [([1, 0, 4], 1), ([1, 5, 4], 3)],
    2: [([2, 3, 0, 4, 5], 2), ([2, 6, 5], 2)],
    3: [([3, 0, 1, 2, 6], 1), ([3, 7, 6], 3)],
    4: [([4, 5, 6, 7], 1), ([4, 7], 3)],
    5: [([5, 6, 7, 4, 0], 2), ([5, 1, 0], 2)],
    6: [([6, 5, 1], 1), ([6, 2, 1], 3)],
    7: [([7, 4, 0, 1, 2], 1), ([7, 3, 2], 3)],
}


def _build_tables():
    transit = {d: [] for d in range(N_DEV)}
    own = {d: [] for d in range(N_DEV)}
    for j, plist in FLOW_PATHS.items():
        c = 0
        for path, n_chunks in plist:
            for _ in range(n_chunks):
                for g in range(G_PER_CHUNK):
                    own[j].append((path, c, g))
                    for i in range(1, len(path) - 1):
                        transit[path[i]].append((i, j, c, g, tuple(path)))
                c += 1
        assert c == 4, (j, c)
    slot = {}
    for d in range(N_DEV):
        transit[d].sort()
        assert len(transit[d]) == N_FWD, (d, len(transit[d]))
        for k, (i, j, c, g, _path) in enumerate(transit[d]):
            slot[(d, j, c, g, i)] = k

    own_dst = np.zeros((N_DEV, N_OWN), np.int32)
    own_row = np.zeros((N_DEV, N_OWN), np.int32)
    own_out = np.zeros((N_DEV, N_OWN), np.int32)
    own_idx = np.zeros((N_DEV, N_OWN), np.int32)
    for d in range(N_DEV):
        assert len(own[d]) == N_OWN
        for k, (path, c, g) in enumerate(own[d]):
            row = (c * G_PER_CHUNK + g) * GR
            nxt = path[1]
            own_dst[d, k] = nxt
            own_row[d, k] = row
            if len(path) == 2:
                own_out[d, k] = 1
                own_idx[d, k] = row // GR
            else:
                own_idx[d, k] = slot[(nxt, d, c, g, 1)]

    fwd_dst = np.zeros((N_DEV, N_FWD), np.int32)
    fwd_out = np.zeros((N_DEV, N_FWD), np.int32)
    fwd_idx = np.zeros((N_DEV, N_FWD), np.int32)
    for d in range(N_DEV):
        for k, (i, j, c, g, path) in enumerate(transit[d]):
            nxt = path[i + 1]
            fwd_dst[d, k] = nxt
            if i + 1 == len(path) - 1:
                fwd_out[d, k] = 1
                fwd_idx[d, k] = c * G_PER_CHUNK + g
            else:
                fwd_idx[d, k] = slot[(nxt, j, c, g, i + 1)]
    return own_dst, own_row, own_out, own_idx, fwd_dst, fwd_out, fwd_idx


_TABLES = _build_tables()


def kernel(x, pi):
    def body(x_ref, pi_ref, own_dst_r, own_row_r, own_out_r, own_idx_r,
             fwd_dst_r, fwd_out_r, fwd_idx_r, out_ref,
             scratch, s_own, s_fwd, t_sem, f_sem):
        my = lax.axis_index("i")

        bar = pltpu.get_barrier_semaphore()
        for p in range(N_DEV):
            @pl.when(my != p)
            def _():
                pl.semaphore_signal(
                    bar, inc=1,
                    device_id=(p,), device_id_type=pl.DeviceIdType.MESH,
                )
        pl.semaphore_wait(bar, N_DEV - 1)

        fast = jnp.bool_(True)
        for j in range(N_DEV):
            fast = jnp.logical_and(fast, pi_ref[j] == int(EXPECTED_PI[j]))

        @pl.when(fast)
        def _fast_path():
            for k in range(N_OWN):
                dst = own_dst_r[my, k]
                row = pl.multiple_of(own_row_r[my, k], GR)
                to_out = own_out_r[my, k]
                idx = own_idx_r[my, k]
                src = x_ref.at[0, pl.ds(row, GR), :]

                @pl.when(to_out == 1)
                def _():
                    pltpu.make_async_remote_copy(
                        src_ref=src,
                        dst_ref=out_ref.at[0, pl.ds(pl.multiple_of(idx * GR, GR), GR), :],
                        send_sem=s_own.at[k],
                        recv_sem=f_sem.at[idx],
                        device_id=(dst,),
                        device_id_type=pl.DeviceIdType.MESH,
                    ).start()

                @pl.when(to_out == 0)
                def _():
                    pltpu.make_async_remote_copy(
                        src_ref=src,
                        dst_ref=scratch.at[idx],
                        send_sem=s_own.at[k],
                        recv_sem=t_sem.at[idx],
                        device_id=(dst,),
                        device_id_type=pl.DeviceIdType.MESH,
                    ).start()

            for k in range(N_FWD):
                pltpu.make_async_remote_copy(
                    src_ref=scratch.at[k],
                    dst_ref=scratch.at[k],
                    send_sem=s_fwd.at[k],
                    recv_sem=t_sem.at[k],
                    device_id=(0,),
                    device_id_type=pl.DeviceIdType.MESH,
                ).wait_recv()

                dst = fwd_dst_r[my, k]
                to_out = fwd_out_r[my, k]
                idx = fwd_idx_r[my, k]

                @pl.when(to_out == 1)
                def _():
                    pltpu.make_async_remote_copy(
                        src_ref=scratch.at[k],
                        dst_ref=out_ref.at[0, pl.ds(pl.multiple_of(idx * GR, GR), GR), :],
                        send_sem=s_fwd.at[k],
                        recv_sem=f_sem.at[idx],
                        device_id=(dst,),
                        device_id_type=pl.DeviceIdType.MESH,
                    ).start()

                @pl.when(to_out == 0)
                def _():
                    pltpu.make_async_remote_copy(
                        src_ref=scratch.at[k],
                        dst_ref=scratch.at[idx],
                        send_sem=s_fwd.at[k],
                        recv_sem=t_sem.at[idx],
                        device_id=(dst,),
                        device_id_type=pl.DeviceIdType.MESH,
                    ).start()

            for k in range(N_OWN):
                pltpu.make_async_remote_copy(
                    src_ref=scratch.at[0],
                    dst_ref=out_ref.at[0, pl.ds(k * GR, GR), :],
                    send_sem=s_own.at[0],
                    recv_sem=f_sem.at[k],
                    device_id=(0,),
                    device_id_type=pl.DeviceIdType.MESH,
                ).wait_recv()

            for k in range(N_OWN):
                row = pl.multiple_of(own_row_r[my, k], GR)
                pltpu.make_async_remote_copy(
                    src_ref=x_ref.at[0, pl.ds(row, GR), :],
                    dst_ref=scratch.at[0],
                    send_sem=s_own.at[k],
                    recv_sem=t_sem.at[0],
                    device_id=(0,),
                    device_id_type=pl.DeviceIdType.MESH,
                ).wait_send()
            for k in range(N_FWD):
                pltpu.make_async_remote_copy(
                    src_ref=scratch.at[k],
                    dst_ref=scratch.at[0],
                    send_sem=s_fwd.at[k],
                    recv_sem=t_sem.at[0],
                    device_id=(0,),
                    device_id_type=pl.DeviceIdType.MESH,
                ).wait_send()

        @pl.when(jnp.logical_not(fast))
        def _fallback():
            dst = pi_ref[my]
            rdma = pltpu.make_async_remote_copy(
                src_ref=x_ref.at[0],
                dst_ref=out_ref.at[0],
                send_sem=s_own.at[0],
                recv_sem=f_sem.at[0],
                device_id=(dst,),
                device_id_type=pl.DeviceIdType.MESH,
            )
            rdma.start()
            rdma.wait()

    own_dst, own_row, own_out, own_idx, fwd_dst, fwd_out, fwd_idx = _TABLES
    out_shape = jax.ShapeDtypeStruct(x.shape, jnp.float32)
    smem = pl.BlockSpec(memory_space=pltpu.SMEM)
    return pl.pallas_call(
        body,
        out_shape=out_shape,
        in_specs=[pl.BlockSpec(memory_space=pl.ANY)] + [smem] * 8,
        out_specs=pl.BlockSpec(memory_space=pl.ANY),
        scratch_shapes=[
            pltpu.VMEM((N_FWD, GR, 2048), jnp.float32),
            pltpu.SemaphoreType.DMA((N_OWN,)),
            pltpu.SemaphoreType.DMA((N_FWD,)),
            pltpu.SemaphoreType.DMA((N_FWD,)),
            pltpu.SemaphoreType.DMA((N_OWN,)),
        ],
        compiler_params=pltpu.CompilerParams(
            collective_id=0, vmem_limit_bytes=64 * 1024 * 1024
        ),
    )(x, jnp.asarray(pi, jnp.int32),
      jnp.asarray(own_dst), jnp.asarray(own_row), jnp.asarray(own_out),
      jnp.asarray(own_idx), jnp.asarray(fwd_dst), jnp.asarray(fwd_out),
      jnp.asarray(fwd_idx))


# device time: 314370 ns/iter; 1.6137x vs baseline; 1.6137x over previous
import numpy as np

import jax
import jax.numpy as jnp
from jax import lax
from jax.experimental import pallas as pl
from jax.experimental.pallas import tpu as pltpu

N_DEV = 8
EXPECTED_PI = np.array([3, 4, 5, 6, 7, 0, 1, 2], dtype=np.int32)

GR = 128
G_PER_CHUNK = 8
N_OWN = 32
N_FWD = 40

FLOW_PATHS = {
    0: [([0, 1, 2, 3], 1), ([0, 3], 3)],
    1: [([1, 0, 4], 1), ([1, 5, 4], 3)],
    2: [([2, 3, 0, 4, 5], 2), ([2, 6, 5], 2)],
    3: [([3, 0, 1, 2, 6], 1), ([3, 7, 6], 3)],
    4: [([4, 5, 6, 7], 1), ([4, 7], 3)],
    5: [([5, 6, 7, 4, 0], 2), ([5, 1, 0], 2)],
    6: [([6, 5, 1], 1), ([6, 2, 1], 3)],
    7: [([7, 4, 0, 1, 2], 1), ([7, 3, 2], 3)],
}


def _simulate(own, transit_order):
    wire = GR * 2048 * 4 / 93e9 * 1e6
    hop, issue = 2.0, 0.1
    link_free, arrival = {}, {}
    fwd_t = [0.0] * N_DEV
    fwd_n = [0] * N_DEV

    def send(t, a, b, key):
        t1 = max(t, link_free.get((a, b), 0.0)) + wire
        link_free[(a, b)] = t1
        arrival[key] = t1 + hop

    for d in range(N_DEV):
        t = 0.0
        for path, c, g in own[d]:
            nxt = path[1]
            key = ((nxt, 'F', c * G_PER_CHUNK + g) if len(path) == 2
                   else (nxt, d, c, g, 1))
            send(t, d, nxt, key)
            t += issue
    progress = True
    while progress:
        progress = False
        for d in range(N_DEV):
            while fwd_n[d] < len(transit_order[d]):
                i, j, c, g, path = transit_order[d][fwd_n[d]]
                key = (d, j, c, g, i)
                if key not in arrival:
                    break
                t = max(arrival[key], fwd_t[d])
                nxt = path[i + 1]
                okey = ((nxt, 'F', c * G_PER_CHUNK + g)
                        if i + 1 == len(path) - 1
                        else (nxt, j, c, g, i + 1))
                send(t, d, nxt, okey)
                fwd_t[d] = t + issue
                fwd_n[d] += 1
                progress = True
    assert all(fwd_n[d] == len(transit_order[d]) for d in range(N_DEV))
    return arrival


def _build_tables():
    transit = {d: [] for d in range(N_DEV)}
    own = {d: [] for d in range(N_DEV)}
    for j, plist in FLOW_PATHS.items():
        c = 0
        for path, n_chunks in plist:
            for _ in range(n_chunks):
                for g in range(G_PER_CHUNK):
                    own[j].append((tuple(path), c, g))
                    for i in range(1, len(path) - 1):
                        transit[path[i]].append((i, j, c, g, tuple(path)))
                c += 1
        assert c == 4, (j, c)
    order = {d: sorted(transit[d]) for d in range(N_DEV)}
    for _ in range(12):
        arrival = _simulate(own, order)
        order = {
            d: sorted(order[d],
                      key=lambda e: arrival[(d, e[1], e[2], e[3], e[0])])
            for d in range(N_DEV)
        }
    slot = {}
    for d in range(N_DEV):
        transit[d] = order[d]
        assert len(transit[d]) == N_FWD, (d, len(transit[d]))
        for k, (i, j, c, g, _path) in enumerate(transit[d]):
            slot[(d, j, c, g, i)] = k

    own_dst = np.zeros((N_DEV, N_OWN), np.int32)
    own_row = np.zeros((N_DEV, N_OWN), np.int32)
    own_out = np.zeros((N_DEV, N_OWN), np.int32)
    own_idx = np.zeros((N_DEV, N_OWN), np.int32)
    for d in range(N_DEV):
        assert len(own[d]) == N_OWN
        for k, (path, c, g) in enumerate(own[d]):
            row = (c * G_PER_CHUNK + g) * GR
            nxt = path[1]
            own_dst[d, k] = nxt
            own_row[d, k] = row
            if len(path) == 2:
                own_out[d, k] = 1
                own_idx[d, k] = row // GR
            else:
                own_idx[d, k] = slot[(nxt, d, c, g, 1)]

    fwd_dst = np.zeros((N_DEV, N_FWD), np.int32)
    fwd_out = np.zeros((N_DEV, N_FWD), np.int32)
    fwd_idx = np.zeros((N_DEV, N_FWD), np.int32)
    for d in range(N_DEV):
        for k, (i, j, c, g, path) in enumerate(transit[d]):
            nxt = path[i + 1]
            fwd_dst[d, k] = nxt
            if i + 1 == len(path) - 1:
                fwd_out[d, k] = 1
                fwd_idx[d, k] = c * G_PER_CHUNK + g
            else:
                fwd_idx[d, k] = slot[(nxt, j, c, g, i + 1)]
    return own_dst, own_row, own_out, own_idx, fwd_dst, fwd_out, fwd_idx


_TABLES = _build_tables()


def kernel(x, pi):
    def body(x_ref, pi_ref, own_dst_r, own_row_r, own_out_r, own_idx_r,
             fwd_dst_r, fwd_out_r, fwd_idx_r, out_ref,
             scratch, s_own, s_fwd, t_sem, f_sem):
        my = lax.axis_index("i")

        bar = pltpu.get_barrier_semaphore()
        for p in range(N_DEV):
            @pl.when(my != p)
            def _():
                pl.semaphore_signal(
                    bar, inc=1,
                    device_id=(p,), device_id_type=pl.DeviceIdType.MESH,
                )
        pl.semaphore_wait(bar, N_DEV - 1)

        fast = jnp.bool_(True)
        for j in range(N_DEV):
            fast = jnp.logical_and(fast, pi_ref[j] == int(EXPECTED_PI[j]))

        @pl.when(fast)
        def _fast_path():
            for k in range(N_OWN):
                dst = own_dst_r[my, k]
                row = pl.multiple_of(own_row_r[my, k], GR)
                to_out = own_out_r[my, k]
                idx = own_idx_r[my, k]
                src = x_ref.at[0, pl.ds(row, GR), :]

                @pl.when(to_out == 1)
                def _():
                    pltpu.make_async_remote_copy(
                        src_ref=src,
                        dst_ref=out_ref.at[0, pl.ds(pl.multiple_of(idx * GR, GR), GR), :],
                        send_sem=s_own.at[k],
                        recv_sem=f_sem.at[idx],
                        device_id=(dst,),
                        device_id_type=pl.DeviceIdType.MESH,
                    ).start()

                @pl.when(to_out == 0)
                def _():
                    pltpu.make_async_remote_copy(
                        src_ref=src,
                        dst_ref=scratch.at[idx],
                        send_sem=s_own.at[k],
                        recv_sem=t_sem.at[idx],
                        device_id=(dst,),
                        device_id_type=pl.DeviceIdType.MESH,
                    ).start()

            for k in range(N_FWD):
                pltpu.make_async_remote_copy(
                    src_ref=scratch.at[k],
                    dst_ref=scratch.at[k],
                    send_sem=s_fwd.at[k],
                    recv_sem=t_sem.at[k],
                    device_id=(0,),
                    device_id_type=pl.DeviceIdType.MESH,
                ).wait_recv()

                dst = fwd_dst_r[my, k]
                to_out = fwd_out_r[my, k]
                idx = fwd_idx_r[my, k]

                @pl.when(to_out == 1)
                def _():
                    pltpu.make_async_remote_copy(
                        src_ref=scratch.at[k],
                        dst_ref=out_ref.at[0, pl.ds(pl.multiple_of(idx * GR, GR), GR), :],
                        send_sem=s_fwd.at[k],
                        recv_sem=f_sem.at[idx],
                        device_id=(dst,),
                        device_id_type=pl.DeviceIdType.MESH,
                    ).start()

                @pl.when(to_out == 0)
                def _():
                    pltpu.make_async_remote_copy(
                        src_ref=scratch.at[k],
                        dst_ref=scratch.at[idx],
                        send_sem=s_fwd.at[k],
                        recv_sem=t_sem.at[idx],
                        device_id=(dst,),
                        device_id_type=pl.DeviceIdType.MESH,
                    ).start()

            for k in range(N_OWN):
                pltpu.make_async_remote_copy(
                    src_ref=scratch.at[0],
                    dst_ref=out_ref.at[0, pl.ds(k * GR, GR), :],
                    send_sem=s_own.at[0],
                    recv_sem=f_sem.at[k],
                    device_id=(0,),
                    device_id_type=pl.DeviceIdType.MESH,
                ).wait_recv()

            for k in range(N_OWN):
                row = pl.multiple_of(own_row_r[my, k], GR)
                pltpu.make_async_remote_copy(
                    src_ref=x_ref.at[0, pl.ds(row, GR), :],
                    dst_ref=scratch.at[0],
                    send_sem=s_own.at[k],
                    recv_sem=t_sem.at[0],
                    device_id=(0,),
                    device_id_type=pl.DeviceIdType.MESH,
                ).wait_send()
            for k in range(N_FWD):
                pltpu.make_async_remote_copy(
                    src_ref=scratch.at[k],
                    dst_ref=scratch.at[0],
                    send_sem=s_fwd.at[k],
                    recv_sem=t_sem.at[0],
                    device_id=(0,),
                    device_id_type=pl.DeviceIdType.MESH,
                ).wait_send()

        @pl.when(jnp.logical_not(fast))
        def _fallback():
            dst = pi_ref[my]
            rdma = pltpu.make_async_remote_copy(
                src_ref=x_ref.at[0],
                dst_ref=out_ref.at[0],
                send_sem=s_own.at[0],
                recv_sem=f_sem.at[0],
                device_id=(dst,),
                device_id_type=pl.DeviceIdType.MESH,
            )
            rdma.start()
            rdma.wait()

    own_dst, own_row, own_out, own_idx, fwd_dst, fwd_out, fwd_idx = _TABLES
    out_shape = jax.ShapeDtypeStruct(x.shape, jnp.float32)
    smem = pl.BlockSpec(memory_space=pltpu.SMEM)
    return pl.pallas_call(
        body,
        out_shape=out_shape,
        in_specs=[pl.BlockSpec(memory_space=pl.ANY)] + [smem] * 8,
        out_specs=pl.BlockSpec(memory_space=pl.ANY),
        scratch_shapes=[
            pltpu.VMEM((N_FWD, GR, 2048), jnp.float32),
            pltpu.SemaphoreType.DMA((N_OWN,)),
            pltpu.SemaphoreType.DMA((N_FWD,)),
            pltpu.SemaphoreType.DMA((N_FWD,)),
            pltpu.SemaphoreType.DMA((N_OWN,)),
        ],
        compiler_params=pltpu.CompilerParams(
            collective_id=0, vmem_limit_bytes=64 * 1024 * 1024
        ),
    )(x, jnp.asarray(pi, jnp.int32),
      jnp.asarray(own_dst), jnp.asarray(own_row), jnp.asarray(own_out),
      jnp.asarray(own_idx), jnp.asarray(fwd_dst), jnp.asarray(fwd_out),
      jnp.asarray(fwd_idx))


# device time: 310125 ns/iter; 1.6358x vs baseline; 1.0137x over previous
import numpy as np

import jax
import jax.numpy as jnp
from jax import lax
from jax.experimental import pallas as pl
from jax.experimental.pallas import tpu as pltpu

N_DEV = 8
EXPECTED_PI = np.array([3, 4, 5, 6, 7, 0, 1, 2], dtype=np.int32)

GR = 64
G_PER_CHUNK = 16
N_OWN = 64
N_FWD = 80

FLOW_PATHS = {
    0: [([0, 1, 2, 3], 1), ([0, 3], 3)],
    1: [([1, 0, 4], 1), ([1, 5, 4], 3)],
    2: [([2, 3, 0, 4, 5], 2), ([2, 6, 5], 2)],
    3: [([3, 0, 1, 2, 6], 1), ([3, 7, 6], 3)],
    4: [([4, 5, 6, 7], 1), ([4, 7], 3)],
    5: [([5, 6, 7, 4, 0], 2), ([5, 1, 0], 2)],
    6: [([6, 5, 1], 1), ([6, 2, 1], 3)],
    7: [([7, 4, 0, 1, 2], 1), ([7, 3, 2], 3)],
}


def _simulate(own, transit_order):
    wire = GR * 2048 * 4 / 93e9 * 1e6
    hop, issue = 2.0, 0.1
    link_free, arrival = {}, {}
    fwd_t = [0.0] * N_DEV
    fwd_n = [0] * N_DEV

    def send(t, a, b, key):
        t1 = max(t, link_free.get((a, b), 0.0)) + wire
        link_free[(a, b)] = t1
        arrival[key] = t1 + hop

    for d in range(N_DEV):
        t = 0.0
        for path, c, g in own[d]:
            nxt = path[1]
            key = ((nxt, 'F', c * G_PER_CHUNK + g) if len(path) == 2
                   else (nxt, d, c, g, 1))
            send(t, d, nxt, key)
            t += issue
    progress = True
    while progress:
        progress = False
        for d in range(N_DEV):
            while fwd_n[d] < len(transit_order[d]):
                i, j, c, g, path = transit_order[d][fwd_n[d]]
                key = (d, j, c, g, i)
                if key not in arrival:
                    break
                t = max(arrival[key], fwd_t[d])
                nxt = path[i + 1]
                okey = ((nxt, 'F', c * G_PER_CHUNK + g)
                        if i + 1 == len(path) - 1
                        else (nxt, j, c, g, i + 1))
                send(t, d, nxt, okey)
                fwd_t[d] = t + issue
                fwd_n[d] += 1
                progress = True
    assert all(fwd_n[d] == len(transit_order[d]) for d in range(N_DEV))
    return arrival


def _build_tables():
    transit = {d: [] for d in range(N_DEV)}
    own = {d: [] for d in range(N_DEV)}
    for j, plist in FLOW_PATHS.items():
        c = 0
        for path, n_chunks in plist:
            for _ in range(n_chunks):
                for g in range(G_PER_CHUNK):
                    own[j].append((tuple(path), c, g))
                    for i in range(1, len(path) - 1):
                        transit[path[i]].append((i, j, c, g, tuple(path)))
                c += 1
        assert c == 4, (j, c)
    order = {d: sorted(transit[d]) for d in range(N_DEV)}
    for _ in range(12):
        arrival = _simulate(own, order)
        order = {
            d: sorted(order[d],
                      key=lambda e: arrival[(d, e[1], e[2], e[3], e[0])])
            for d in range(N_DEV)
        }
    slot = {}
    for d in range(N_DEV):
        transit[d] = order[d]
        assert len(transit[d]) == N_FWD, (d, len(transit[d]))
        for k, (i, j, c, g, _path) in enumerate(transit[d]):
            slot[(d, j, c, g, i)] = k

    own_dst = np.zeros((N_DEV, N_OWN), np.int32)
    own_row = np.zeros((N_DEV, N_OWN), np.int32)
    own_out = np.zeros((N_DEV, N_OWN), np.int32)
    own_idx = np.zeros((N_DEV, N_OWN), np.int32)
    for d in range(N_DEV):
        assert len(own[d]) == N_OWN
        for k, (path, c, g) in enumerate(own[d]):
            row = (c * G_PER_CHUNK + g) * GR
            nxt = path[1]
            own_dst[d, k] = nxt
            own_row[d, k] = row
            if len(path) == 2:
                own_out[d, k] = 1
                own_idx[d, k] = row // GR
            else:
                own_idx[d, k] = slot[(nxt, d, c, g, 1)]

    fwd_dst = np.zeros((N_DEV, N_FWD), np.int32)
    fwd_out = np.zeros((N_DEV, N_FWD), np.int32)
    fwd_idx = np.zeros((N_DEV, N_FWD), np.int32)
    for d in range(N_DEV):
        for k, (i, j, c, g, path) in enumerate(transit[d]):
            nxt = path[i + 1]
            fwd_dst[d, k] = nxt
            if i + 1 == len(path) - 1:
                fwd_out[d, k] = 1
                fwd_idx[d, k] = c * G_PER_CHUNK + g
            else:
                fwd_idx[d, k] = slot[(nxt, j, c, g, i + 1)]
    return own_dst, own_row, own_out, own_idx, fwd_dst, fwd_out, fwd_idx


_TABLES = _build_tables()


def kernel(x, pi):
    def body(x_ref, pi_ref, own_dst_r, own_row_r, own_out_r, own_idx_r,
             fwd_dst_r, fwd_out_r, fwd_idx_r, out_ref,
             scratch, s_own, s_fwd, t_sem, f_sem):
        my = lax.axis_index("i")

        bar = pltpu.get_barrier_semaphore()
        for p in range(N_DEV):
            @pl.when(my != p)
            def _():
                pl.semaphore_signal(
                    bar, inc=1,
                    device_id=(p,), device_id_type=pl.DeviceIdType.MESH,
                )
        pl.semaphore_wait(bar, N_DEV - 1)

        fast = jnp.bool_(True)
        for j in range(N_DEV):
            fast = jnp.logical_and(fast, pi_ref[j] == int(EXPECTED_PI[j]))

        @pl.when(fast)
        def _fast_path():
            for k in range(N_OWN):
                dst = own_dst_r[my, k]
                row = pl.multiple_of(own_row_r[my, k], GR)
                to_out = own_out_r[my, k]
                idx = own_idx_r[my, k]
                src = x_ref.at[0, pl.ds(row, GR), :]

                @pl.when(to_out == 1)
                def _():
                    pltpu.make_async_remote_copy(
                        src_ref=src,
                        dst_ref=out_ref.at[0, pl.ds(pl.multiple_of(idx * GR, GR), GR), :],
                        send_sem=s_own.at[k],
                        recv_sem=f_sem.at[idx],
                        device_id=(dst,),
                        device_id_type=pl.DeviceIdType.MESH,
                    ).start()

                @pl.when(to_out == 0)
                def _():
                    pltpu.make_async_remote_copy(
                        src_ref=src,
                        dst_ref=scratch.at[idx],
                        send_sem=s_own.at[k],
                        recv_sem=t_sem.at[idx],
                        device_id=(dst,),
                        device_id_type=pl.DeviceIdType.MESH,
                    ).start()

            for k in range(N_FWD):
                pltpu.make_async_remote_copy(
                    src_ref=scratch.at[k],
                    dst_ref=scratch.at[k],
                    send_sem=s_fwd.at[k],
                    recv_sem=t_sem.at[k],
                    device_id=(0,),
                    device_id_type=pl.DeviceIdType.MESH,
                ).wait_recv()

                dst = fwd_dst_r[my, k]
                to_out = fwd_out_r[my, k]
                idx = fwd_idx_r[my, k]

                @pl.when(to_out == 1)
                def _():
                    pltpu.make_async_remote_copy(
                        src_ref=scratch.at[k],
                        dst_ref=out_ref.at[0, pl.ds(pl.multiple_of(idx * GR, GR), GR), :],
                        send_sem=s_fwd.at[k],
                        recv_sem=f_sem.at[idx],
                        device_id=(dst,),
                        device_id_type=pl.DeviceIdType.MESH,
                    ).start()

                @pl.when(to_out == 0)
                def _():
                    pltpu.make_async_remote_copy(
                        src_ref=scratch.at[k],
                        dst_ref=scratch.at[idx],
                        send_sem=s_fwd.at[k],
                        recv_sem=t_sem.at[idx],
                        device_id=(dst,),
                        device_id_type=pl.DeviceIdType.MESH,
                    ).start()

            for k in range(N_OWN):
                pltpu.make_async_remote_copy(
                    src_ref=scratch.at[0],
                    dst_ref=out_ref.at[0, pl.ds(k * GR, GR), :],
                    send_sem=s_own.at[0],
                    recv_sem=f_sem.at[k],
                    device_id=(0,),
                    device_id_type=pl.DeviceIdType.MESH,
                ).wait_recv()

            for k in range(N_OWN):
                row = pl.multiple_of(own_row_r[my, k], GR)
                pltpu.make_async_remote_copy(
                    src_ref=x_ref.at[0, pl.ds(row, GR), :],
                    dst_ref=scratch.at[0],
                    send_sem=s_own.at[k],
                    recv_sem=t_sem.at[0],
                    device_id=(0,),
                    device_id_type=pl.DeviceIdType.MESH,
                ).wait_send()
            for k in range(N_FWD):
                pltpu.make_async_remote_copy(
                    src_ref=scratch.at[k],
                    dst_ref=scratch.at[0],
                    send_sem=s_fwd.at[k],
                    recv_sem=t_sem.at[0],
                    device_id=(0,),
                    device_id_type=pl.DeviceIdType.MESH,
                ).wait_send()

        @pl.when(jnp.logical_not(fast))
        def _fallback():
            dst = pi_ref[my]
            rdma = pltpu.make_async_remote_copy(
                src_ref=x_ref.at[0],
                dst_ref=out_ref.at[0],
                send_sem=s_own.at[0],
                recv_sem=f_sem.at[0],
                device_id=(dst,),
                device_id_type=pl.DeviceIdType.MESH,
            )
            rdma.start()
            rdma.wait()

    own_dst, own_row, own_out, own_idx, fwd_dst, fwd_out, fwd_idx = _TABLES
    out_shape = jax.ShapeDtypeStruct(x.shape, jnp.float32)
    smem = pl.BlockSpec(memory_space=pltpu.SMEM)
    return pl.pallas_call(
        body,
        out_shape=out_shape,
        in_specs=[pl.BlockSpec(memory_space=pl.ANY)] + [smem] * 8,
        out_specs=pl.BlockSpec(memory_space=pl.ANY),
        scratch_shapes=[
            pltpu.VMEM((N_FWD, GR, 2048), jnp.float32),
            pltpu.SemaphoreType.DMA((N_OWN,)),
            pltpu.SemaphoreType.DMA((N_FWD,)),
            pltpu.SemaphoreType.DMA((N_FWD,)),
            pltpu.SemaphoreType.DMA((N_OWN,)),
        ],
        compiler_params=pltpu.CompilerParams(
            collective_id=0, vmem_limit_bytes=64 * 1024 * 1024
        ),
    )(x, jnp.asarray(pi, jnp.int32),
      jnp.asarray(own_dst), jnp.asarray(own_row), jnp.asarray(own_out),
      jnp.asarray(own_idx), jnp.asarray(fwd_dst), jnp.asarray(fwd_out),
      jnp.asarray(fwd_idx))
